# Initial kernel scaffold; baseline (speedup 1.0000x reference)
import jax
import jax.numpy as jnp
from jax.experimental import pallas as pl


def kernel(pn_feats, pillar_pixels, W_b, b_b, W_h, b_h):
    raise NotImplementedError("write your pallas kernel here")


# speedup vs baseline: 1.6128x; 1.6128x over previous
import jax
import jax.numpy as jnp
from jax import lax
from jax.experimental import pallas as pl
from jax.experimental.pallas import tpu as pltpu
from jax.experimental.pallas import tpu_sc as plsc

N, P, C, S = 8, 12000, 64, 256
CW = 128
PS = S + 2
PIX = PS * PS
ZLEN = 66576
DUMP0 = 66568
BSTRIDE = 66672
IMSZ = 66576
SENT = 66568
NPASS = 4
L = 16
ZCH = 192
ZPT = 22192
NZCH = ZPT // ZCH
ZTAIL = ZPT - NZCH * ZCH
FCH = 96
HC = 32
MROW = HC * PS
NB = S // HC
XR = 8784


def _sc_build(rows_hbm, cols_hbm, feats_hbm, out_hbm,
              zbuf, r_ref, c_ref, idxmap, fbuf, dl_ref, sem0):
  cid = lax.axis_index("c")
  sid = lax.axis_index("s")
  iota = lax.iota(jnp.int32, L)
  owner = sid < 4

  def _pix(i_base):
    rv = r_ref[pl.ds(i_base, L)]
    cv = c_ref[pl.ds(i_base, L)]
    valid = (rv >= 0) & (rv < S) & (cv >= 0) & (cv < S)
    q = jnp.where(valid, (rv + 1) * PS + (cv + 1), SENT)
    return q, valid

  @pl.when(jnp.logical_not(owner))
  def _zero_fill():
    zeros16 = jnp.zeros((L,), jnp.float32)

    def zb(r, _):
      for k in range(CW // L):
        zbuf[r, pl.ds(k * L, L)] = zeros16
      return 0
    lax.fori_loop(0, ZCH, zb, 0)

    j = sid - 4
    zn = cid * 4 + j // 3
    zbase = zn * BSTRIDE + (j % 3) * ZPT

    def zf(k, _):
      d0 = out_hbm.at[pl.ds(zbase + (2 * k) * ZCH, ZCH)]
      d1 = out_hbm.at[pl.ds(zbase + (2 * k + 1) * ZCH, ZCH)]
      a = pltpu.async_copy(zbuf, d0, sem0)
      b = pltpu.async_copy(zbuf, d1, sem0)
      a.wait()
      b.wait()
      return 0
    lax.fori_loop(0, NZCH // 2, zf, 0)
    pltpu.sync_copy(zbuf, out_hbm.at[pl.ds(zbase + (NZCH - 1) * ZCH, ZCH)])
    pltpu.sync_copy(zbuf.at[pl.ds(0, ZTAIL)],
                    out_hbm.at[pl.ds(zbase + NZCH * ZCH, ZTAIL)])

  @pl.when(owner)
  def _winners():
    nn = cid * 4 + sid
    pltpu.sync_copy(rows_hbm.at[pl.ds(nn * P, P)], r_ref)
    pltpu.sync_copy(cols_hbm.at[pl.ds(nn * P, P)], c_ref)
    neg1 = jnp.full((L,), -1, jnp.int32)

    def ms(i, _):
      idxmap[pl.ds(i * L, L)] = neg1
      return 0
    lax.fori_loop(0, IMSZ // L, ms, 0)

    def p1(i, _):
      q, _v = _pix(i * L)
      plsc.store_scatter(idxmap, [q], iota + i * L)
      return 0
    lax.fori_loop(0, P // L, p1, 0)

    def pk(i, _):
      q, _v = _pix(i * L)
      pv = iota + i * L
      cur = plsc.load_gather(idxmap, [q])
      plsc.store_scatter(idxmap, [q], pv, mask=pv > cur)
      return 0
    for _ in range(NPASS - 1):
      lax.fori_loop(0, P // L, pk, 0)

  plsc.subcore_barrier()

  @pl.when(owner)
  def _scatter_rows():
    nn = cid * 4 + sid
    base_out = nn * BSTRIDE

    def ch(j, _):
      pltpu.sync_copy(feats_hbm.at[pl.ds(nn * P + j * FCH, FCH)], fbuf)
      for i in range(FCH // L):
        q, valid = _pix(j * FCH + i * L)
        pv = iota + (j * FCH + i * L)
        cur = plsc.load_gather(idxmap, [q])
        win = valid & (cur == pv)
        dest = jnp.where(win, base_out + q, base_out + DUMP0 + (pv & 63))
        dl_ref[pl.ds(i * L, L)] = dest
      pltpu.async_copy(fbuf, out_hbm.at[dl_ref], sem0).wait()
      return 0
    lax.fori_loop(0, P // FCH, ch, 0)


def _build_pseudo(rows2, cols2, feats128):
  f = pl.kernel(
      _sc_build,
      out_type=jax.ShapeDtypeStruct((N * BSTRIDE, CW), jnp.float32),
      mesh=plsc.VectorSubcoreMesh(core_axis_name="c", subcore_axis_name="s",
                                  num_cores=2, num_subcores=16),
      scratch_types=[
          pltpu.VMEM((ZCH, CW), jnp.float32),
          pltpu.VMEM((P,), jnp.int32),
          pltpu.VMEM((P,), jnp.int32),
          pltpu.VMEM((IMSZ,), jnp.int32),
          pltpu.VMEM((FCH, CW), jnp.float32),
          pltpu.VMEM((FCH,), jnp.int32),
          pltpu.SemaphoreType.DMA,
      ],
      compiler_params=pltpu.CompilerParams(needs_layout_passes=False),
  )
  return f(rows2, cols2, feats128)


def _tc_conv(ps_ref, w_ref, bb_ref, wh_ref, bh_ref, out_ref, xbuf, sem):
  n = pl.program_id(0)
  rb = pl.program_id(1)
  base = n * BSTRIDE + rb * HC * PS
  pltpu.sync_copy(ps_ref.at[pl.ds(base, XR)], xbuf)
  bb = bb_ref[:].reshape(1, C)
  bh = bh_ref[0]
  acc = jnp.zeros((MROW, C), jnp.float32)
  for s9 in range(9):
    off = (s9 // 3) * PS + (s9 % 3)
    x = xbuf[pl.ds(off, MROW), :]
    acc = acc + lax.dot_general(x, w_ref[s9], (((1,), (0,)), ((), ())),
                                preferred_element_type=jnp.float32)
  y = jnp.maximum(acc + bb, 0.0)
  z = lax.dot_general(y, wh_ref[:], (((1,), (0,)), ((), ())),
                      preferred_element_type=jnp.float32)
  out_ref[0, :, :] = z.reshape(HC, PS)[:, :S] + bh


def _conv(pseudo, w9, bb, wh2, bh1):
  return pl.pallas_call(
      _tc_conv,
      grid=(N, NB),
      in_specs=[
          pl.BlockSpec(memory_space=pl.ANY),
          pl.BlockSpec((9, CW, C), lambda n, rb: (0, 0, 0)),
          pl.BlockSpec((C,), lambda n, rb: (0,)),
          pl.BlockSpec((C, 1), lambda n, rb: (0, 0)),
          pl.BlockSpec((1,), lambda n, rb: (0,)),
      ],
      out_specs=pl.BlockSpec((1, HC, S), lambda n, rb: (n, rb, 0)),
      out_shape=jax.ShapeDtypeStruct((N, S, S), jnp.float32),
      scratch_shapes=[
          pltpu.VMEM((XR, CW), jnp.float32),
          pltpu.SemaphoreType.DMA,
      ],
  )(pseudo, w9, bb, wh2, bh1)


def kernel(pn_feats, pillar_pixels, W_b, b_b, W_h, b_h):
  rows2 = pillar_pixels[..., 0].astype(jnp.int32).reshape(N * P)
  cols2 = pillar_pixels[..., 1].astype(jnp.int32).reshape(N * P)
  feats128 = jnp.pad(pn_feats.reshape(N * P, C), ((0, 0), (0, CW - C)))
  pseudo = _build_pseudo(rows2, cols2, feats128)
  w9 = jnp.transpose(W_b, (2, 3, 1, 0)).reshape(9, C, C)
  w9 = jnp.pad(w9, ((0, 0), (0, CW - C), (0, 0)))
  wh2 = W_h.reshape(C, 1)
  bh1 = b_h.reshape(1)
  return _conv(pseudo, w9, b_b.astype(jnp.float32), wh2, bh1)


# speedup vs baseline: 1.9779x; 1.2264x over previous
import jax
import jax.numpy as jnp
from jax import lax
from jax.experimental import pallas as pl
from jax.experimental.pallas import tpu as pltpu
from jax.experimental.pallas import tpu_sc as plsc

N, P, C, S = 8, 12000, 64, 256
CW = 128
PS = S + 2
PIX = PS * PS
ZLEN = 66576
DUMP0 = 66568
BSTRIDE = 66672
IMSZ = 66576
SENT = 66568
NPASS = 4
L = 16
ZCH = 112
ZPT = 22192
NZCH = ZPT // ZCH
ZTAIL = ZPT - NZCH * ZCH
FCH = 96
HC = 32
MROW = HC * PS
NB = S // HC
XR = 8784


def _sc_build(rows_hbm, cols_hbm, feats_hbm, out_hbm,
              zbuf, r_ref, c_ref, idxmap, fbuf, dl_ref, sem0, semf):
  cid = lax.axis_index("c")
  sid = lax.axis_index("s")
  iota = lax.iota(jnp.int32, L)
  owner = sid < 4

  def _pix(i_base):
    rv = r_ref[pl.ds(i_base, L)]
    cv = c_ref[pl.ds(i_base, L)]
    valid = (rv >= 0) & (rv < S) & (cv >= 0) & (cv < S)
    q = jnp.where(valid, (rv + 1) * PS + (cv + 1), SENT)
    return q, valid

  @pl.when(jnp.logical_not(owner))
  def _zero_fill():
    zeros16 = jnp.zeros((L,), jnp.float32)

    def zb(r, _):
      for k in range(CW // L):
        zbuf[r, pl.ds(k * L, L)] = zeros16
      return 0
    lax.fori_loop(0, ZCH, zb, 0)

    j = sid - 4
    zn = cid * 4 + j // 3
    zbase = zn * BSTRIDE + (j % 3) * ZPT

    def zf(k, _):
      d0 = out_hbm.at[pl.ds(zbase + (2 * k) * ZCH, ZCH)]
      d1 = out_hbm.at[pl.ds(zbase + (2 * k + 1) * ZCH, ZCH)]
      a = pltpu.async_copy(zbuf, d0, sem0.at[0])
      b = pltpu.async_copy(zbuf, d1, sem0.at[1])
      a.wait()
      b.wait()
      return 0
    lax.fori_loop(0, NZCH // 2, zf, 0)
    pltpu.sync_copy(zbuf.at[pl.ds(0, ZTAIL)],
                    out_hbm.at[pl.ds(zbase + NZCH * ZCH, ZTAIL)])

  @pl.when(owner)
  def _winners():
    nn = cid * 4 + sid
    pltpu.sync_copy(rows_hbm.at[pl.ds(nn * P, P)], r_ref)
    pltpu.sync_copy(cols_hbm.at[pl.ds(nn * P, P)], c_ref)
    neg1 = jnp.full((L,), -1, jnp.int32)

    def ms(i, _):
      idxmap[pl.ds(i * L, L)] = neg1
      return 0
    lax.fori_loop(0, IMSZ // L, ms, 0)

    def p1(i, _):
      q, _v = _pix(i * L)
      plsc.store_scatter(idxmap, [q], iota + i * L)
      return 0
    lax.fori_loop(0, P // L, p1, 0)

    def pk(i, _):
      q, _v = _pix(i * L)
      pv = iota + i * L
      cur = plsc.load_gather(idxmap, [q])
      plsc.store_scatter(idxmap, [q], pv, mask=pv > cur)
      return 0
    for _ in range(NPASS - 1):
      lax.fori_loop(0, P // L, pk, 0)

  plsc.subcore_barrier()

  @pl.when(owner)
  def _scatter_rows():
    nn = cid * 4 + sid
    base_out = nn * BSTRIDE
    NCH = P // FCH

    def fin(j, slot):
      return pltpu.make_async_copy(
          feats_hbm.at[pl.ds(nn * P + j * FCH, FCH)], fbuf.at[slot], semf.at[slot])

    def fout(j, slot):
      return pltpu.make_async_copy(
          fbuf.at[slot], out_hbm.at[dl_ref.at[slot]], sem0.at[slot])

    fin(0, 0).start()

    def ch(j, _):
      slot = lax.rem(j, 2)
      nxt = 1 - slot
      fin(j, slot).wait()
      for i in range(FCH // L):
        q, valid = _pix(j * FCH + i * L)
        pv = iota + (j * FCH + i * L)
        cur = plsc.load_gather(idxmap, [q])
        win = valid & (cur == pv)
        dest = jnp.where(win, base_out + q, base_out + DUMP0 + (pv & 63))
        dl_ref[slot, pl.ds(i * L, L)] = dest

      @pl.when(j > 0)
      def _():
        fout(j - 1, nxt).wait()

      @pl.when(j < NCH - 1)
      def _():
        fin(j + 1, nxt).start()
      fout(j, slot).start()
      return 0
    lax.fori_loop(0, NCH, ch, 0)
    fout(NCH - 1, lax.rem(NCH - 1, 2)).wait()


def _build_pseudo(rows2, cols2, feats128):
  f = pl.kernel(
      _sc_build,
      out_type=jax.ShapeDtypeStruct((N * BSTRIDE, CW), jnp.float32),
      mesh=plsc.VectorSubcoreMesh(core_axis_name="c", subcore_axis_name="s",
                                  num_cores=2, num_subcores=16),
      scratch_types=[
          pltpu.VMEM((ZCH, CW), jnp.float32),
          pltpu.VMEM((P,), jnp.int32),
          pltpu.VMEM((P,), jnp.int32),
          pltpu.VMEM((IMSZ,), jnp.int32),
          pltpu.VMEM((2, FCH, CW), jnp.float32),
          pltpu.VMEM((2, FCH), jnp.int32),
          pltpu.SemaphoreType.DMA((2,)),
          pltpu.SemaphoreType.DMA((2,)),
      ],
      compiler_params=pltpu.CompilerParams(needs_layout_passes=False),
  )
  return f(rows2, cols2, feats128)


def _tc_conv(ps_ref, w_ref, bb_ref, wh_ref, bh_ref, out_ref, xb, xbh, sem):
  n = pl.program_id(0)
  rb = pl.program_id(1)
  g = n * NB + rb
  slot = lax.rem(g, 2)
  nxt = 1 - slot

  def stage(gg, ss):
    nn2 = gg // NB
    rb2 = gg - nn2 * NB
    b2 = nn2 * BSTRIDE + rb2 * HC * PS
    return pltpu.make_async_copy(ps_ref.at[pl.ds(b2, XR)], xb.at[ss],
                                 sem.at[ss])

  @pl.when(g == 0)
  def _():
    stage(0, 0).start()

  @pl.when(g + 1 < N * NB)
  def _():
    stage(g + 1, nxt).start()
  stage(g, slot).wait()

  for t in range(6):
    xbh[pl.ds(t * (XR // 6), XR // 6), :] = (
        xb[slot, pl.ds(t * (XR // 6), XR // 6), :].astype(jnp.bfloat16))

  bb = bb_ref[:].reshape(1, C)
  bh = bh_ref[0]
  acc = jnp.zeros((MROW, C), jnp.float32)
  for s9 in range(9):
    off = (s9 // 3) * PS + (s9 % 3)
    x = xbh[pl.ds(off, MROW), :]
    acc = acc + lax.dot_general(x, w_ref[s9], (((1,), (0,)), ((), ())),
                                preferred_element_type=jnp.float32)
  y = jnp.maximum(acc + bb, 0.0)
  z = lax.dot_general(y, wh_ref[:], (((1,), (0,)), ((), ())),
                      preferred_element_type=jnp.float32)
  out_ref[0, :, :] = z.reshape(HC, PS)[:, :S] + bh


def _conv(pseudo, w9, bb, wh2, bh1):
  return pl.pallas_call(
      _tc_conv,
      grid=(N, NB),
      in_specs=[
          pl.BlockSpec(memory_space=pl.ANY),
          pl.BlockSpec((9, CW, C), lambda n, rb: (0, 0, 0)),
          pl.BlockSpec((C,), lambda n, rb: (0,)),
          pl.BlockSpec((C, 1), lambda n, rb: (0, 0)),
          pl.BlockSpec((1,), lambda n, rb: (0,)),
      ],
      out_specs=pl.BlockSpec((1, HC, S), lambda n, rb: (n, rb, 0)),
      out_shape=jax.ShapeDtypeStruct((N, S, S), jnp.float32),
      scratch_shapes=[
          pltpu.VMEM((2, XR, CW), jnp.float32),
          pltpu.VMEM((XR, CW), jnp.bfloat16),
          pltpu.SemaphoreType.DMA((2,)),
      ],
  )(pseudo, w9, bb, wh2, bh1)


def kernel(pn_feats, pillar_pixels, W_b, b_b, W_h, b_h):
  rows2 = pillar_pixels[..., 0].astype(jnp.int32).reshape(N * P)
  cols2 = pillar_pixels[..., 1].astype(jnp.int32).reshape(N * P)
  feats128 = jnp.pad(pn_feats.reshape(N * P, C), ((0, 0), (0, CW - C)))
  pseudo = _build_pseudo(rows2, cols2, feats128)
  w9 = jnp.transpose(W_b, (2, 3, 1, 0)).reshape(9, C, C)
  w9 = jnp.pad(w9, ((0, 0), (0, CW - C), (0, 0))).astype(jnp.bfloat16)
  wh2 = W_h.reshape(C, 1)
  bh1 = b_h.reshape(1)
  return _conv(pseudo, w9, b_b.astype(jnp.float32), wh2, bh1)
